# baseline (speedup 1.0000x reference)
import dataclasses
import functools

import jax
import jax.numpy as jnp
from jax import lax
from jax.experimental import pallas as pl
from jax.experimental.pallas import tpu as pltpu
from jax.experimental.pallas import tpu_sc as plsc

N_USERS = 1000000
N_ITEMS = 100000
EMB_DIM = 128
BATCH = 16384
MAX_SCORE = 5.0

NUM_CORES = 2
NUM_SUBCORES = 16
NUM_WORKERS = NUM_CORES * NUM_SUBCORES
BPW = BATCH // NUM_WORKERS

CHUNK = 64
NPC = BPW // CHUNK
LANES = 16
GROUPS = EMB_DIM // LANES


def _sc_fused(users, items, user_emb, item_emb, user_bias, item_bias,
              global_bias):
    mesh = plsc.VectorSubcoreMesh(core_axis_name="c", subcore_axis_name="s")
    cp = pltpu.CompilerParams()
    if "needs_layout_passes" in pltpu.CompilerParams.__dataclass_fields__:
        cp = dataclasses.replace(cp, needs_layout_passes=False)

    @functools.partial(
        pl.kernel,
        compiler_params=cp,
        out_type=(
            jax.ShapeDtypeStruct((BATCH,), jnp.float32),
            jax.ShapeDtypeStruct((BATCH, EMB_DIM), jnp.float32),
            jax.ShapeDtypeStruct((BATCH, EMB_DIM), jnp.float32),
            jax.ShapeDtypeStruct((BATCH,), jnp.float32),
            jax.ShapeDtypeStruct((BATCH,), jnp.float32),
        ),
        mesh=mesh,
        scratch_types=[
            pltpu.VMEM((BPW,), jnp.int32),
            pltpu.VMEM((BPW,), jnp.int32),
            pltpu.VMEM((7, CHUNK, EMB_DIM), jnp.float32),
            pltpu.VMEM((7, CHUNK, EMB_DIM), jnp.float32),
            pltpu.VMEM((BPW,), jnp.float32),
            pltpu.VMEM((BPW,), jnp.float32),
            pltpu.VMEM((CHUNK, LANES), jnp.float32),
            pltpu.VMEM((BPW,), jnp.float32),
            pltpu.VMEM((LANES,), jnp.float32),
            pltpu.SemaphoreType.DMA((7,)),
            pltpu.SemaphoreType.DMA((7,)),
            pltpu.SemaphoreType.DMA((7,)),
            pltpu.SemaphoreType.DMA((7,)),
            pltpu.SemaphoreType.DMA,
            pltpu.SemaphoreType.DMA,
        ],
    )
    def k(users_hbm, items_hbm, uemb_hbm, iemb_hbm, ubias_hbm, ibias_hbm,
          gb_hbm, sc_out, ue_out, ie_out, ub_out, ib_out,
          idx_u, idx_i, ru, ri, bu_v, bi_v, tmp_v, sc_v, gb_v,
          sgu, sgi, swu, swi, sb, sgb):
        wid = lax.axis_index("s") * NUM_CORES + lax.axis_index("c")
        base = wid * BPW
        sl = pl.ds(base, BPW)
        ld_u = pltpu.make_async_copy(users_hbm.at[sl], idx_u, sgu.at[0])
        ld_i = pltpu.make_async_copy(items_hbm.at[sl], idx_i, sgi.at[0])
        ld_u.start()
        ld_i.start()
        ld_u.wait()
        ld_i.wait()
        bg_g = pltpu.make_async_copy(
            gb_hbm.at[jnp.zeros((LANES,), jnp.int32)], gb_v, sgb)
        bg_g.start()
        bg_u = pltpu.make_async_copy(ubias_hbm.at[idx_u], bu_v, sb)
        bg_i = pltpu.make_async_copy(ibias_hbm.at[idx_i], bi_v, sb)
        bg_u.start()
        bg_i.start()

        def gather(tbl, c):
            p = c % 7
            src, idxr, sem, ring = (
                (uemb_hbm, idx_u, sgu, ru) if tbl == 0
                else (iemb_hbm, idx_i, sgi, ri))
            return pltpu.make_async_copy(
                src.at[idxr.at[pl.ds(c * CHUNK, CHUNK)]], ring.at[p],
                sem.at[p])

        def writeback(tbl, c):
            p = c % 7
            dst, sem, ring = ((ue_out, swu, ru) if tbl == 0
                              else (ie_out, swi, ri))
            return pltpu.make_async_copy(
                ring.at[p], dst.at[pl.ds(base + c * CHUNK, CHUNK)],
                sem.at[p])

        iota = lax.iota(jnp.int32, LANES)

        def compute_chunk(c):
            p = c % 7
            ru_s = ru.at[p]
            ri_s = ri.at[p]

            @plsc.parallel_loop(0, CHUNK)
            def _(r):
                acc = (ru_s[r, pl.ds(0, LANES)]
                       * ri_s[r, pl.ds(0, LANES)])
                for g in range(1, GROUPS):
                    cs = pl.ds(g * LANES, LANES)
                    acc += ru_s[r, cs] * ri_s[r, cs]
                tmp_v[r, :] = acc

            for t in range(CHUNK // LANES):
                rows = iota + t * LANES
                s16 = plsc.load_gather(
                    tmp_v, [rows, jnp.zeros((LANES,), jnp.int32)])
                for l in range(1, LANES):
                    s16 += plsc.load_gather(
                        tmp_v, [rows, jnp.full((LANES,), l, jnp.int32)])
                q = pl.ds(c * CHUNK + t * LANES, LANES)
                z = s16 + bu_v[q] + bi_v[q] + gb_v[...]
                sc_v[q] = MAX_SCORE / (1.0 + jnp.exp(-z))

        for c in range(min(7, NPC)):
            gather(0, c).start()
            gather(1, c).start()
        for c in range(NPC):
            gather(0, c).wait()
            gather(1, c).wait()
            writeback(0, c).start()
            writeback(1, c).start()
            if c == 0:
                bg_u.wait()
                bg_i.wait()
                bg_g.wait()
            compute_chunk(c)
            if c + 7 < NPC:
                writeback(0, c).wait()
                writeback(1, c).wait()
                gather(0, c + 7).start()
                gather(1, c + 7).start()
        for c in range(max(0, NPC - 7), NPC):
            writeback(0, c).wait()
            writeback(1, c).wait()
        pltpu.sync_copy(sc_v, sc_out.at[sl])
        pltpu.sync_copy(bu_v, ub_out.at[sl])
        pltpu.sync_copy(bi_v, ib_out.at[sl])

    return k(users, items, user_emb, item_emb, user_bias, item_bias,
             global_bias)


@jax.jit
def kernel(users, items, user_emb, item_emb, user_bias, item_bias, global_bias):
    users = users.astype(jnp.int32)
    items = items.astype(jnp.int32)
    pred_score, users_emb, items_emb, users_bias, items_bias = _sc_fused(
        users, items, user_emb, item_emb, user_bias, item_bias, global_bias)
    return (pred_score, users_emb, items_emb, users_bias, items_bias)


# speedup vs baseline: 1.0307x; 1.0307x over previous
import dataclasses
import functools

import jax
import jax.numpy as jnp
from jax import lax
from jax.experimental import pallas as pl
from jax.experimental.pallas import tpu as pltpu
from jax.experimental.pallas import tpu_sc as plsc

N_USERS = 1000000
N_ITEMS = 100000
EMB_DIM = 128
BATCH = 16384
MAX_SCORE = 5.0

NUM_CORES = 2
NUM_SUBCORES = 16
NUM_WORKERS = NUM_CORES * NUM_SUBCORES
BPW = BATCH // NUM_WORKERS

CHUNK = 64
NPC = BPW // CHUNK
LANES = 16
GROUPS = EMB_DIM // LANES


def _sc_fused(users, items, user_emb, item_emb, user_bias, item_bias,
              global_bias):
    mesh = plsc.VectorSubcoreMesh(core_axis_name="c", subcore_axis_name="s")
    cp = pltpu.CompilerParams()
    if "needs_layout_passes" in pltpu.CompilerParams.__dataclass_fields__:
        cp = dataclasses.replace(cp, needs_layout_passes=False)

    @functools.partial(
        pl.kernel,
        compiler_params=cp,
        out_type=(
            jax.ShapeDtypeStruct((BATCH,), jnp.float32),
            jax.ShapeDtypeStruct((BATCH, EMB_DIM), jnp.float32),
            jax.ShapeDtypeStruct((BATCH, EMB_DIM), jnp.float32),
            jax.ShapeDtypeStruct((BATCH,), jnp.float32),
            jax.ShapeDtypeStruct((BATCH,), jnp.float32),
        ),
        mesh=mesh,
        scratch_types=[
            pltpu.VMEM((BPW,), jnp.int32),
            pltpu.VMEM((BPW,), jnp.int32),
            pltpu.VMEM((5, CHUNK, EMB_DIM), jnp.float32),
            pltpu.VMEM((5, CHUNK, EMB_DIM), jnp.float32),
            pltpu.VMEM((BPW,), jnp.float32),
            pltpu.VMEM((BPW,), jnp.float32),
            pltpu.VMEM((CHUNK, LANES), jnp.float32),
            pltpu.VMEM((BPW,), jnp.float32),
            pltpu.VMEM((LANES,), jnp.float32),
            pltpu.SemaphoreType.DMA((5,)),
            pltpu.SemaphoreType.DMA((5,)),
            pltpu.SemaphoreType.DMA((5,)),
            pltpu.SemaphoreType.DMA((5,)),
            pltpu.SemaphoreType.DMA,
            pltpu.SemaphoreType.DMA,
        ],
    )
    def k(users_hbm, items_hbm, uemb_hbm, iemb_hbm, ubias_hbm, ibias_hbm,
          gb_hbm, sc_out, ue_out, ie_out, ub_out, ib_out,
          idx_u, idx_i, ru, ri, bu_v, bi_v, tmp_v, sc_v, gb_v,
          sgu, sgi, swu, swi, sb, sgb):
        wid = lax.axis_index("s") * NUM_CORES + lax.axis_index("c")
        base = wid * BPW
        sl = pl.ds(base, BPW)
        ld_u = pltpu.make_async_copy(users_hbm.at[sl], idx_u, sgu.at[0])
        ld_i = pltpu.make_async_copy(items_hbm.at[sl], idx_i, sgi.at[0])
        ld_u.start()
        ld_i.start()
        ld_u.wait()
        ld_i.wait()
        bg_g = pltpu.make_async_copy(
            gb_hbm.at[jnp.zeros((LANES,), jnp.int32)], gb_v, sgb)
        bg_g.start()
        bg_u = pltpu.make_async_copy(ubias_hbm.at[idx_u], bu_v, sb)
        bg_i = pltpu.make_async_copy(ibias_hbm.at[idx_i], bi_v, sb)
        bg_u.start()
        bg_i.start()

        def gather(tbl, c):
            p = c % 5
            src, idxr, sem, ring = (
                (uemb_hbm, idx_u, sgu, ru) if tbl == 0
                else (iemb_hbm, idx_i, sgi, ri))
            return pltpu.make_async_copy(
                src.at[idxr.at[pl.ds(c * CHUNK, CHUNK)]], ring.at[p],
                sem.at[p])

        def writeback(tbl, c):
            p = c % 5
            dst, sem, ring = ((ue_out, swu, ru) if tbl == 0
                              else (ie_out, swi, ri))
            return pltpu.make_async_copy(
                ring.at[p], dst.at[pl.ds(base + c * CHUNK, CHUNK)],
                sem.at[p])

        iota = lax.iota(jnp.int32, LANES)

        def compute_chunk(c):
            p = c % 5
            ru_s = ru.at[p]
            ri_s = ri.at[p]

            @plsc.parallel_loop(0, CHUNK, unroll=2)
            def _(r):
                acc = (ru_s[r, pl.ds(0, LANES)]
                       * ri_s[r, pl.ds(0, LANES)])
                for g in range(1, GROUPS):
                    cs = pl.ds(g * LANES, LANES)
                    acc += ru_s[r, cs] * ri_s[r, cs]
                tmp_v[r, :] = acc

            for t in range(CHUNK // LANES):
                rows = iota + t * LANES
                s16 = plsc.load_gather(
                    tmp_v, [rows, jnp.zeros((LANES,), jnp.int32)])
                for l in range(1, LANES):
                    s16 += plsc.load_gather(
                        tmp_v, [rows, jnp.full((LANES,), l, jnp.int32)])
                q = pl.ds(c * CHUNK + t * LANES, LANES)
                z = s16 + bu_v[q] + bi_v[q] + gb_v[...]
                sc_v[q] = MAX_SCORE / (1.0 + jnp.exp(-z))

        for c in range(min(5, NPC)):
            gather(0, c).start()
            gather(1, c).start()
        for c in range(NPC):
            gather(0, c).wait()
            gather(1, c).wait()
            writeback(0, c).start()
            writeback(1, c).start()
            if c == 0:
                bg_u.wait()
                bg_i.wait()
                bg_g.wait()
            compute_chunk(c)
            if c + 5 < NPC:
                writeback(0, c).wait()
                writeback(1, c).wait()
                gather(0, c + 5).start()
                gather(1, c + 5).start()
        for c in range(max(0, NPC - 5), NPC):
            writeback(0, c).wait()
            writeback(1, c).wait()
        pltpu.sync_copy(sc_v, sc_out.at[sl])
        pltpu.sync_copy(bu_v, ub_out.at[sl])
        pltpu.sync_copy(bi_v, ib_out.at[sl])

    return k(users, items, user_emb, item_emb, user_bias, item_bias,
             global_bias)


@jax.jit
def kernel(users, items, user_emb, item_emb, user_bias, item_bias, global_bias):
    users = users.astype(jnp.int32)
    items = items.astype(jnp.int32)
    pred_score, users_emb, items_emb, users_bias, items_bias = _sc_fused(
        users, items, user_emb, item_emb, user_bias, item_bias, global_bias)
    return (pred_score, users_emb, items_emb, users_bias, items_bias)


# speedup vs baseline: 1.0383x; 1.0075x over previous
import dataclasses
import functools

import jax
import jax.numpy as jnp
from jax import lax
from jax.experimental import pallas as pl
from jax.experimental.pallas import tpu as pltpu
from jax.experimental.pallas import tpu_sc as plsc

N_USERS = 1000000
N_ITEMS = 100000
EMB_DIM = 128
BATCH = 16384
MAX_SCORE = 5.0

NUM_CORES = 2
NUM_SUBCORES = 16
NUM_WORKERS = NUM_CORES * NUM_SUBCORES
BPW = BATCH // NUM_WORKERS

CHUNK = 64
NPC = BPW // CHUNK
LANES = 16
GROUPS = EMB_DIM // LANES


def _sc_fused(users, items, user_emb, item_emb, user_bias, item_bias,
              global_bias):
    mesh = plsc.VectorSubcoreMesh(core_axis_name="c", subcore_axis_name="s")
    cp = pltpu.CompilerParams()
    if "needs_layout_passes" in pltpu.CompilerParams.__dataclass_fields__:
        cp = dataclasses.replace(cp, needs_layout_passes=False)

    @functools.partial(
        pl.kernel,
        compiler_params=cp,
        out_type=(
            jax.ShapeDtypeStruct((BATCH,), jnp.float32),
            jax.ShapeDtypeStruct((BATCH, EMB_DIM), jnp.float32),
            jax.ShapeDtypeStruct((BATCH, EMB_DIM), jnp.float32),
            jax.ShapeDtypeStruct((BATCH,), jnp.float32),
            jax.ShapeDtypeStruct((BATCH,), jnp.float32),
        ),
        mesh=mesh,
        scratch_types=[
            pltpu.VMEM((BPW,), jnp.int32),
            pltpu.VMEM((BPW,), jnp.int32),
            pltpu.VMEM((5, CHUNK, EMB_DIM), jnp.float32),
            pltpu.VMEM((5, CHUNK, EMB_DIM), jnp.float32),
            pltpu.VMEM((BPW,), jnp.float32),
            pltpu.VMEM((BPW,), jnp.float32),
            pltpu.VMEM((CHUNK, LANES), jnp.float32),
            pltpu.VMEM((BPW,), jnp.float32),
            pltpu.VMEM((LANES,), jnp.float32),
            pltpu.SemaphoreType.DMA((5,)),
            pltpu.SemaphoreType.DMA((5,)),
            pltpu.SemaphoreType.DMA((5,)),
            pltpu.SemaphoreType.DMA((5,)),
            pltpu.SemaphoreType.DMA,
            pltpu.SemaphoreType.DMA,
        ],
    )
    def k(users_hbm, items_hbm, uemb_hbm, iemb_hbm, ubias_hbm, ibias_hbm,
          gb_hbm, sc_out, ue_out, ie_out, ub_out, ib_out,
          idx_u, idx_i, ru, ri, bu_v, bi_v, tmp_v, sc_v, gb_v,
          sgu, sgi, swu, swi, sb, sgb):
        wid = lax.axis_index("s") * NUM_CORES + lax.axis_index("c")
        base = wid * BPW
        sl = pl.ds(base, BPW)
        ld_u = pltpu.make_async_copy(users_hbm.at[sl], idx_u, sgu.at[0])
        ld_i = pltpu.make_async_copy(items_hbm.at[sl], idx_i, sgi.at[0])
        ld_u.start()
        ld_i.start()
        ld_u.wait()
        ld_i.wait()
        bg_g = pltpu.make_async_copy(
            gb_hbm.at[jnp.zeros((LANES,), jnp.int32)], gb_v, sgb)
        bg_g.start()
        bg_u = pltpu.make_async_copy(ubias_hbm.at[idx_u], bu_v, sb)
        bg_i = pltpu.make_async_copy(ibias_hbm.at[idx_i], bi_v, sb)
        bg_u.start()
        bg_i.start()

        def gather(tbl, c):
            p = c % 5
            src, idxr, sem, ring = (
                (uemb_hbm, idx_u, sgu, ru) if tbl == 0
                else (iemb_hbm, idx_i, sgi, ri))
            return pltpu.make_async_copy(
                src.at[idxr.at[pl.ds(c * CHUNK, CHUNK)]], ring.at[p],
                sem.at[p])

        def writeback(tbl, c):
            p = c % 5
            dst, sem, ring = ((ue_out, swu, ru) if tbl == 0
                              else (ie_out, swi, ri))
            return pltpu.make_async_copy(
                ring.at[p], dst.at[pl.ds(base + c * CHUNK, CHUNK)],
                sem.at[p])

        iota = lax.iota(jnp.int32, LANES)

        def compute_chunk(c):
            p = c % 5
            ru_s = ru.at[p]
            ri_s = ri.at[p]

            @plsc.parallel_loop(0, CHUNK)
            def _(r):
                acc = (ru_s[r, pl.ds(0, LANES)]
                       * ri_s[r, pl.ds(0, LANES)])
                for g in range(1, GROUPS):
                    cs = pl.ds(g * LANES, LANES)
                    acc += ru_s[r, cs] * ri_s[r, cs]
                tmp_v[r, :] = acc

            for t in range(CHUNK // LANES):
                rows = iota + t * LANES
                s16 = plsc.load_gather(
                    tmp_v, [rows, jnp.zeros((LANES,), jnp.int32)])
                for l in range(1, LANES):
                    s16 += plsc.load_gather(
                        tmp_v, [rows, jnp.full((LANES,), l, jnp.int32)])
                q = pl.ds(c * CHUNK + t * LANES, LANES)
                z = s16 + bu_v[q] + bi_v[q] + gb_v[...]
                sc_v[q] = MAX_SCORE / (1.0 + jnp.exp(-z))

        for c in range(min(5, NPC)):
            gather(0, c).start()
            gather(1, c).start()
        for c in range(NPC):
            gather(0, c).wait()
            gather(1, c).wait()
            writeback(0, c).start()
            writeback(1, c).start()
            if c == 0:
                bg_u.wait()
                bg_i.wait()
                bg_g.wait()
            compute_chunk(c)
            if c + 5 < NPC:
                writeback(0, c).wait()
                writeback(1, c).wait()
                gather(0, c + 5).start()
                gather(1, c + 5).start()
        for c in range(max(0, NPC - 5), NPC):
            writeback(0, c).wait()
            writeback(1, c).wait()
        out_s = pltpu.make_async_copy(sc_v, sc_out.at[sl], sb)
        out_u = pltpu.make_async_copy(bu_v, ub_out.at[sl], sgb)
        out_i = pltpu.make_async_copy(bi_v, ib_out.at[sl], sb)
        out_s.start()
        out_u.start()
        out_i.start()
        out_s.wait()
        out_u.wait()
        out_i.wait()

    return k(users, items, user_emb, item_emb, user_bias, item_bias,
             global_bias)


@jax.jit
def kernel(users, items, user_emb, item_emb, user_bias, item_bias, global_bias):
    users = users.astype(jnp.int32)
    items = items.astype(jnp.int32)
    pred_score, users_emb, items_emb, users_bias, items_bias = _sc_fused(
        users, items, user_emb, item_emb, user_bias, item_bias, global_bias)
    return (pred_score, users_emb, items_emb, users_bias, items_bias)


# speedup vs baseline: 1.0498x; 1.0110x over previous
import dataclasses
import functools

import jax
import jax.numpy as jnp
from jax import lax
from jax.experimental import pallas as pl
from jax.experimental.pallas import tpu as pltpu
from jax.experimental.pallas import tpu_sc as plsc

N_USERS = 1000000
N_ITEMS = 100000
EMB_DIM = 128
BATCH = 16384
MAX_SCORE = 5.0

NUM_CORES = 2
NUM_SUBCORES = 16
NUM_WORKERS = NUM_CORES * NUM_SUBCORES
BPW = BATCH // NUM_WORKERS

CHUNK = 64
NPC = BPW // CHUNK
LANES = 16
GROUPS = EMB_DIM // LANES


def _sc_fused(users, items, user_emb, item_emb, user_bias, item_bias,
              global_bias):
    mesh = plsc.VectorSubcoreMesh(core_axis_name="c", subcore_axis_name="s")
    cp = pltpu.CompilerParams()
    if "needs_layout_passes" in pltpu.CompilerParams.__dataclass_fields__:
        cp = dataclasses.replace(cp, needs_layout_passes=False)

    @functools.partial(
        pl.kernel,
        compiler_params=cp,
        out_type=(
            jax.ShapeDtypeStruct((BATCH,), jnp.float32),
            jax.ShapeDtypeStruct((BATCH, EMB_DIM), jnp.float32),
            jax.ShapeDtypeStruct((BATCH, EMB_DIM), jnp.float32),
            jax.ShapeDtypeStruct((BATCH,), jnp.float32),
            jax.ShapeDtypeStruct((BATCH,), jnp.float32),
        ),
        mesh=mesh,
        scratch_types=[
            pltpu.VMEM((BPW,), jnp.int32),
            pltpu.VMEM((BPW,), jnp.int32),
            pltpu.VMEM((4, CHUNK, EMB_DIM), jnp.float32),
            pltpu.VMEM((4, CHUNK, EMB_DIM), jnp.float32),
            pltpu.VMEM((BPW,), jnp.float32),
            pltpu.VMEM((BPW,), jnp.float32),
            pltpu.VMEM((CHUNK, LANES), jnp.float32),
            pltpu.VMEM((BPW,), jnp.float32),
            pltpu.VMEM((LANES,), jnp.float32),
            pltpu.SemaphoreType.DMA((4,)),
            pltpu.SemaphoreType.DMA((4,)),
            pltpu.SemaphoreType.DMA((4,)),
            pltpu.SemaphoreType.DMA((4,)),
            pltpu.SemaphoreType.DMA,
            pltpu.SemaphoreType.DMA,
        ],
    )
    def k(users_hbm, items_hbm, uemb_hbm, iemb_hbm, ubias_hbm, ibias_hbm,
          gb_hbm, sc_out, ue_out, ie_out, ub_out, ib_out,
          idx_u, idx_i, ru, ri, bu_v, bi_v, tmp_v, sc_v, gb_v,
          sgu, sgi, swu, swi, sb, sgb):
        wid = lax.axis_index("s") * NUM_CORES + lax.axis_index("c")
        base = wid * BPW
        sl = pl.ds(base, BPW)
        ld_u = pltpu.make_async_copy(users_hbm.at[sl], idx_u, sgu.at[0])
        ld_i = pltpu.make_async_copy(items_hbm.at[sl], idx_i, sgi.at[0])
        ld_u.start()
        ld_i.start()
        ld_u.wait()
        ld_i.wait()
        bg_g = pltpu.make_async_copy(
            gb_hbm.at[jnp.zeros((LANES,), jnp.int32)], gb_v, sgb)
        bg_g.start()
        bg_u = pltpu.make_async_copy(ubias_hbm.at[idx_u], bu_v, sb)
        bg_i = pltpu.make_async_copy(ibias_hbm.at[idx_i], bi_v, sb)
        bg_u.start()
        bg_i.start()

        def gather(tbl, c):
            p = c % 4
            src, idxr, sem, ring = (
                (uemb_hbm, idx_u, sgu, ru) if tbl == 0
                else (iemb_hbm, idx_i, sgi, ri))
            return pltpu.make_async_copy(
                src.at[idxr.at[pl.ds(c * CHUNK, CHUNK)]], ring.at[p],
                sem.at[p])

        def writeback(tbl, c):
            p = c % 4
            dst, sem, ring = ((ue_out, swu, ru) if tbl == 0
                              else (ie_out, swi, ri))
            return pltpu.make_async_copy(
                ring.at[p], dst.at[pl.ds(base + c * CHUNK, CHUNK)],
                sem.at[p])

        iota = lax.iota(jnp.int32, LANES)

        def compute_chunk(c):
            p = c % 4
            ru_s = ru.at[p]
            ri_s = ri.at[p]

            @plsc.parallel_loop(0, CHUNK)
            def _(r):
                acc = (ru_s[r, pl.ds(0, LANES)]
                       * ri_s[r, pl.ds(0, LANES)])
                for g in range(1, GROUPS):
                    cs = pl.ds(g * LANES, LANES)
                    acc += ru_s[r, cs] * ri_s[r, cs]
                tmp_v[r, :] = acc

            for t in range(CHUNK // LANES):
                rows = iota + t * LANES
                s16 = plsc.load_gather(
                    tmp_v, [rows, jnp.zeros((LANES,), jnp.int32)])
                for l in range(1, LANES):
                    s16 += plsc.load_gather(
                        tmp_v, [rows, jnp.full((LANES,), l, jnp.int32)])
                q = pl.ds(c * CHUNK + t * LANES, LANES)
                z = s16 + bu_v[q] + bi_v[q] + gb_v[...]
                sc_v[q] = MAX_SCORE / (1.0 + jnp.exp(-z))

        for c in range(min(4, NPC)):
            gather(0, c).start()
            gather(1, c).start()
        for c in range(NPC):
            gather(0, c).wait()
            gather(1, c).wait()
            writeback(0, c).start()
            writeback(1, c).start()
            if c == 0:
                bg_u.wait()
                bg_i.wait()
                bg_g.wait()
            compute_chunk(c)
            if c + 4 < NPC:
                writeback(0, c).wait()
                writeback(1, c).wait()
                gather(0, c + 4).start()
                gather(1, c + 4).start()
        for c in range(max(0, NPC - 4), NPC):
            writeback(0, c).wait()
            writeback(1, c).wait()
        out_s = pltpu.make_async_copy(sc_v, sc_out.at[sl], sb)
        out_u = pltpu.make_async_copy(bu_v, ub_out.at[sl], sgb)
        out_i = pltpu.make_async_copy(bi_v, ib_out.at[sl], sb)
        out_s.start()
        out_u.start()
        out_i.start()
        out_s.wait()
        out_u.wait()
        out_i.wait()

    return k(users, items, user_emb, item_emb, user_bias, item_bias,
             global_bias)


@jax.jit
def kernel(users, items, user_emb, item_emb, user_bias, item_bias, global_bias):
    users = users.astype(jnp.int32)
    items = items.astype(jnp.int32)
    pred_score, users_emb, items_emb, users_bias, items_bias = _sc_fused(
        users, items, user_emb, item_emb, user_bias, item_bias, global_bias)
    return (pred_score, users_emb, items_emb, users_bias, items_bias)
